# Initial kernel scaffold; baseline (speedup 1.0000x reference)
import jax
import jax.numpy as jnp
from jax.experimental import pallas as pl


def kernel(card_ids, hero_pos, acting_pos, num_players, scalars, bets, action, seq_lengths, card_tab, src_tab, hero_tab, actpos_tab, np_tab, Ws, bs, Wb, bb, Wa, ba, Wc, bc, gamma, beta):
    raise NotImplementedError("write your pallas kernel here")


# speedup vs baseline: 3.2284x; 3.2284x over previous
import functools

import jax
import jax.numpy as jnp
from jax.experimental import pallas as pl

B = 1024
L = 50
D = 64
MP = 9
NA = 16
C = 7

EV_BLK = 800


def _prep_kernel(card_tab_ref, src_tab_ref, hero_tab_ref, actpos_tab_ref,
                 np_tab_ref, Ws_ref, bs_ref, Wb_ref, bb_ref, Wa_ref, ba_ref,
                 Wc_ref, bc_ref, beta_ref,
                 card_proj_ref, wctx_ref, bias_ref, beta7_ref):
    Wc = Wc_ref[...]
    wc_card = Wc[0:D, :]
    wc_hero = Wc[D:2 * D, :]
    wc_act = Wc[2 * D:3 * D, :]
    wc_np = Wc[3 * D:4 * D, :]
    wc_s = Wc[4 * D:5 * D, :]
    wc_b = Wc[5 * D:6 * D, :]
    wc_a = Wc[6 * D:7 * D, :]
    f32 = jnp.float32
    dot = functools.partial(jnp.dot, preferred_element_type=f32,
                            precision=jax.lax.Precision.HIGHEST)
    card_proj_ref[...] = dot(card_tab_ref[...], wc_card)
    wctx_ref[...] = jnp.concatenate([
        dot(Ws_ref[...], wc_s),
        dot(Wb_ref[...], wc_b),
        dot(Wa_ref[...], wc_a),
        dot(hero_tab_ref[...], wc_hero),
        dot(actpos_tab_ref[...], wc_act),
        dot(np_tab_ref[...], wc_np),
    ], axis=0)
    bias_ref[...] = (bc_ref[...] + dot(bs_ref[...], wc_s)
                     + dot(bb_ref[...], wc_b) + dot(ba_ref[...], wc_a))
    src = src_tab_ref[...]
    beta7_ref[...] = beta_ref[...] + jnp.concatenate(
        [jnp.broadcast_to(src[0:1, :], (5, D)),
         jnp.broadcast_to(src[1:2, :], (2, D))], axis=0)


def _main_kernel(card_ids_ref, hero_ref, act_ref, npl_ref, scalars_ref,
                 bets_ref, action_ref, lpos_ref, seq_ref,
                 card_proj_ref, wctx_ref, bias_ref, gamma_ref, beta7_ref,
                 out_ref, mask_ref):
    f32 = jnp.float32
    N = EV_BLK
    ioh = jax.lax.broadcasted_iota(jnp.int32, (N, MP), 1)
    ion = jax.lax.broadcasted_iota(jnp.int32, (N, MP + 1), 1)
    feats = jnp.concatenate([
        scalars_ref[...], bets_ref[...], action_ref[...],
        (hero_ref[...] == ioh).astype(f32),
        (act_ref[...] == ioh).astype(f32),
        (npl_ref[...] == ion).astype(f32),
    ], axis=1)
    ctx = jnp.dot(feats, wctx_ref[...],
                  preferred_element_type=f32) + bias_ref[...]
    m = (lpos_ref[...] < seq_ref[...]).astype(f32)
    mask_ref[...] = jnp.broadcast_to(m, (N, C))
    ioc = jax.lax.broadcasted_iota(jnp.int32, (N, 53), 1)
    gamma = gamma_ref[...]
    card_proj = card_proj_ref[...]
    for c in range(C):
        oh = (card_ids_ref[:, c:c + 1] == ioc).astype(f32)
        h = jnp.dot(oh, card_proj, preferred_element_type=f32) + ctx
        mu = jnp.mean(h, axis=-1, keepdims=True)
        xc = h - mu
        var = jnp.mean(xc * xc, axis=-1, keepdims=True)
        xhat = xc * jax.lax.rsqrt(var + 1e-5)
        h = xhat * gamma + beta7_ref[c:c + 1, :]
        out_ref[:, c, :] = h * m


def kernel(card_ids, hero_pos, acting_pos, num_players, scalars, bets, action,
           seq_lengths, card_tab, src_tab, hero_tab, actpos_tab, np_tab,
           Ws, bs, Wb, bb, Wa, ba, Wc, bc, gamma, beta):
    f32 = jnp.float32
    i32 = jnp.int32
    card_proj, wctx, bias, beta7 = pl.pallas_call(
        _prep_kernel,
        out_shape=(
            jax.ShapeDtypeStruct((53, D), f32),
            jax.ShapeDtypeStruct((55, D), f32),
            jax.ShapeDtypeStruct((1, D), f32),
            jax.ShapeDtypeStruct((C, D), f32),
        ),
    )(card_tab, src_tab, hero_tab, actpos_tab, np_tab,
      Ws, bs.reshape(1, D), Wb, bb.reshape(1, D), Wa, ba.reshape(1, D),
      Wc, bc.reshape(1, D), beta.reshape(1, D))

    BL = B * L
    cid2 = card_ids.astype(i32).reshape(BL, C)
    hero2 = hero_pos.astype(i32).reshape(BL, 1)
    act2 = acting_pos.astype(i32).reshape(BL, 1)
    npl2 = num_players.astype(i32).reshape(BL, 1)
    sc2 = scalars.reshape(BL, 2)
    bt2 = bets.reshape(BL, MP)
    ac2 = action.reshape(BL, NA)
    lpos = jnp.broadcast_to(jnp.arange(L, dtype=i32)[None, :], (B, L)).reshape(BL, 1)
    seq2 = jnp.broadcast_to(seq_lengths.astype(i32)[:, None], (B, L)).reshape(BL, 1)

    grid = (BL // EV_BLK,)
    ev_spec1 = pl.BlockSpec((EV_BLK, 1), lambda i: (i, 0))
    const2 = lambda shape: pl.BlockSpec(shape, lambda i: (0, 0))
    in_specs = [
        pl.BlockSpec((EV_BLK, C), lambda i: (i, 0)),
        ev_spec1, ev_spec1, ev_spec1,
        pl.BlockSpec((EV_BLK, 2), lambda i: (i, 0)),
        pl.BlockSpec((EV_BLK, MP), lambda i: (i, 0)),
        pl.BlockSpec((EV_BLK, NA), lambda i: (i, 0)),
        ev_spec1, ev_spec1,
        const2((53, D)), const2((55, D)), const2((1, D)),
        const2((1, D)), const2((C, D)),
    ]
    out_specs = (
        pl.BlockSpec((EV_BLK, C, D), lambda i: (i, 0, 0)),
        pl.BlockSpec((EV_BLK, C), lambda i: (i, 0)),
    )
    emb, mask = pl.pallas_call(
        _main_kernel,
        grid=grid,
        in_specs=in_specs,
        out_specs=out_specs,
        out_shape=(
            jax.ShapeDtypeStruct((BL, C, D), f32),
            jax.ShapeDtypeStruct((BL, C), f32),
        ),
    )(cid2, hero2, act2, npl2, sc2, bt2, ac2, lpos, seq2,
      card_proj, wctx, bias, gamma.reshape(1, D), beta7)
    return emb.reshape(B, L * C, D), mask.reshape(B, L * C)
